# baseline (speedup 1.0000x reference)
import functools

import jax
import jax.numpy as jnp
from jax import lax
from jax.experimental import pallas as pl
from jax.experimental.pallas import tpu as pltpu
from jax.experimental.pallas import tpu_sc as plsc

_N = 10000
_E = 320000
_NP = 10240
_CHUNK = 80
_NWORK = 32
_EPW = _E // _NWORK
_NCH = _EPW // _CHUNK
_ROWS_PT = _NP // 16


def _make_agg(d, chunk, nbuf, with_counts, async_s=True):
  nch = _EPW // chunk
  rem = _EPW - nch * chunk
  assert rem == 0
  mesh = plsc.VectorSubcoreMesh(core_axis_name="c", subcore_axis_name="s")
  src_1d = chunk % 8 == 0
  out_type = [jax.ShapeDtypeStruct((2, _NP, d), jnp.float32)]
  scratch = [
      pltpu.VMEM((_EPW,), jnp.int32) if src_1d
      else pltpu.VMEM((nch, chunk), jnp.int32),
      pltpu.VMEM((nch, chunk), jnp.int32),
  ]
  scratch += [pltpu.VMEM((chunk, d), jnp.float32) for _ in range(nbuf)]
  scratch += [pltpu.SemaphoreType.DMA for _ in range(2 * nbuf)]
  if with_counts:
    out_type.append(jax.ShapeDtypeStruct((2 * _NP,), jnp.float32))
    scratch += [
        pltpu.VMEM((chunk,), jnp.float32),
        pltpu.VMEM((_ROWS_PT,), jnp.float32),
        pltpu.VMEM_SHARED((_NP,), jnp.float32),
    ]
  scratch.append(pltpu.VMEM_SHARED((_NP, d), jnp.float32))

  def body(x_hbm, src_hbm, dst_hbm, out_hbm, *rest):
    if with_counts:
      cnt_hbm = rest[0]
      rest = rest[1:]
    src_v, dst_v = rest[0], rest[1]
    rows = rest[2:2 + nbuf]
    gsem = rest[2 + nbuf:2 + 2 * nbuf]
    ssem = rest[2 + 2 * nbuf:2 + 3 * nbuf]
    rest = rest[2 + 3 * nbuf:]
    if with_counts:
      ones_v, zcnt_v, cnt_acc = rest[0], rest[1], rest[2]
      rest = rest[3:]
    acc = rest[0]
    cid = lax.axis_index("c")
    sid = lax.axis_index("s")
    wid = sid * 2 + cid
    tid = sid

    if src_1d:
      pltpu.sync_copy(src_hbm.at[pl.ds(wid * _EPW, _EPW)], src_v)
    else:
      pltpu.sync_copy(src_hbm.at[wid], src_v)
    pltpu.sync_copy(dst_hbm.at[wid], dst_v)

    def zrow(r, c):
      for cc in range(d // 16):
        rows[0][r, pl.ds(cc * 16, 16)] = jnp.zeros((16,), jnp.float32)
      return c
    lax.fori_loop(0, chunk, zrow, 0)
    nz = _ROWS_PT // chunk
    for j in range(nz):
      pltpu.sync_copy(
          rows[0], acc.at[pl.ds(tid * _ROWS_PT + j * chunk, chunk)])
    zr = _ROWS_PT - nz * chunk
    if zr:
      pltpu.sync_copy(
          rows[0].at[pl.ds(0, zr)],
          acc.at[pl.ds(tid * _ROWS_PT + nz * chunk, zr)])
    if with_counts:
      def zc(i, c):
        zcnt_v[pl.ds(i * 16, 16)] = jnp.zeros((16,), jnp.float32)
        return c
      lax.fori_loop(0, _ROWS_PT // 16, zc, 0)
      pltpu.sync_copy(zcnt_v, cnt_acc.at[pl.ds(tid * _ROWS_PT, _ROWS_PT)])
      for i in range(chunk // 16):
        ones_v[pl.ds(i * 16, 16)] = jnp.ones((16,), jnp.float32)

    def fire_g(k, q):
      idx = (src_v.at[pl.ds(k * chunk, chunk)] if src_1d else src_v.at[k])
      pltpu.async_copy(x_hbm.at[idx], rows[q], gsem[q])

    def wait_g(q):
      pltpu.make_async_copy(
          x_hbm.at[pl.ds(0, chunk)], rows[q], gsem[q]).wait()

    def fire_s(k, q):
      if async_s:
        pltpu.async_copy(rows[q], acc.at[dst_v.at[k]], ssem[q], add=True)
      else:
        pltpu.sync_copy(rows[q], acc.at[dst_v.at[k]], add=True)
      if with_counts:
        pltpu.sync_copy(ones_v, cnt_acc.at[dst_v.at[k]], add=True)

    def wait_s(q):
      if async_s:
        pltpu.make_async_copy(
            x_hbm.at[pl.ds(0, chunk)], rows[q], ssem[q]).wait()

    for q in range(nbuf):
      fire_g(q, q)
    plsc.subcore_barrier()

    full = nch // nbuf
    def grp(kg, c):
      c0 = kg * nbuf
      for q in range(nbuf):
        wait_g(q)
        fire_s(c0 + q, q)
      for q in range(nbuf):
        nxt = c0 + nbuf + q
        @pl.when(nxt < nch)
        def _():
          wait_s(q)
          fire_g(nxt, q)
      return c
    lax.fori_loop(0, full, grp, 0)
    for q in range(nch - full * nbuf):
      wait_g(q)
      fire_s(full * nbuf + q, q)
    for q in range(nbuf):
      wait_s(q)

    plsc.subcore_barrier()
    row0 = tid * _ROWS_PT
    pltpu.sync_copy(acc.at[pl.ds(row0, _ROWS_PT)],
                    out_hbm.at[cid, pl.ds(row0, _ROWS_PT)])
    if with_counts:
      pltpu.sync_copy(cnt_acc.at[pl.ds(row0, _ROWS_PT)],
                      cnt_hbm.at[pl.ds(cid * _NP + row0, _ROWS_PT)])

  params = pltpu.CompilerParams(use_tc_tiling_on_sc=False) if d == 64 else None
  return pl.kernel(body, out_type=tuple(out_type), mesh=mesh,
                   scratch_types=tuple(scratch), compiler_params=params)


_BN = 2048


def _inv_of(cnt_blk):
  c = cnt_blk[0] + cnt_blk[1]
  return (1.0 / jnp.maximum(c, 1.0))[:, None]


def _dot(a, b):
  return jax.lax.dot_general(a, b, (((1,), (0,)), ((), ())),
                             preferred_element_type=jnp.float32)


def _tc_call(body, out_shapes, in_specs, out_specs):
  return pl.pallas_call(
      body,
      grid=(_NP // _BN,),
      in_specs=in_specs,
      out_specs=out_specs,
      out_shape=out_shapes,
  )


def _spec_rows(d):
  return pl.BlockSpec((_BN, d), lambda i: (i, 0))


def _spec_pair(d):
  return pl.BlockSpec((2, _BN, d), lambda i: (0, i, 0))


def _spec_cnt():
  return pl.BlockSpec((2, _BN), lambda i: (0, i))


def _spec_full(r, c):
  return pl.BlockSpec((r, c), lambda i: (0, 0))


def _stage_a(p0, cnt, x, wl0t, bl0, wr0t, wl1t):
  def body(p_ref, c_ref, x_ref, wl_ref, b_ref, wr_ref, w1_ref, h0_ref, t1_ref):
    m = (p_ref[0] + p_ref[1]) * _inv_of(c_ref)
    h0 = jnp.maximum(
        _dot(m, wl_ref[...]) + b_ref[...] + _dot(x_ref[...], wr_ref[...]), 0.0)
    h0_ref[...] = h0
    t1_ref[...] = _dot(h0, w1_ref[...])
  return _tc_call(
      body,
      (jax.ShapeDtypeStruct((_NP, 128), jnp.float32),
       jax.ShapeDtypeStruct((_NP, 64), jnp.float32)),
      [_spec_pair(128), _spec_cnt(), _spec_rows(128), _spec_full(128, 128),
       _spec_full(1, 128), _spec_full(128, 128), _spec_full(128, 64)],
      (_spec_rows(128), _spec_rows(64)),
  )(p0, cnt, x, wl0t, bl0, wr0t, wl1t)


def _stage_b(p1, cnt, h0, bl1, wr1t):
  def body(p_ref, c_ref, h_ref, b_ref, wr_ref, o_ref):
    m = (p_ref[0] + p_ref[1]) * _inv_of(c_ref)
    o_ref[...] = m + b_ref[...] + _dot(h_ref[...], wr_ref[...])
  return _tc_call(
      body,
      jax.ShapeDtypeStruct((_NP, 64), jnp.float32),
      [_spec_pair(64), _spec_cnt(), _spec_rows(128), _spec_full(1, 64),
       _spec_full(128, 64)],
      _spec_rows(64),
  )(p1, cnt, h0, bl1, wr1t)


def _stage_c(p2, cnt, h1, wl2t, bl2, wr2t):
  def body(p_ref, c_ref, h_ref, wl_ref, b_ref, wr_ref, o_ref):
    m = (p_ref[0] + p_ref[1]) * _inv_of(c_ref)
    o_ref[...] = jnp.maximum(
        _dot(m, wl_ref[...]) + b_ref[...] + _dot(h_ref[...], wr_ref[...]), 0.0)
  return _tc_call(
      body,
      jax.ShapeDtypeStruct((_NP, 128), jnp.float32),
      [_spec_pair(64), _spec_cnt(), _spec_rows(64), _spec_full(64, 128),
       _spec_full(1, 128), _spec_full(64, 128)],
      _spec_rows(128),
  )(p2, cnt, h1, wl2t, bl2, wr2t)


def _stage_d(p3, cnt, h2, wl3t, bl3, wr3t):
  def body(p_ref, c_ref, h_ref, wl_ref, b_ref, wr_ref, o_ref):
    m = (p_ref[0] + p_ref[1]) * _inv_of(c_ref)
    o_ref[...] = (_dot(m, wl_ref[...]) + b_ref[...]
                  + _dot(h_ref[...], wr_ref[...]))
  return _tc_call(
      body,
      jax.ShapeDtypeStruct((_NP, 128), jnp.float32),
      [_spec_pair(128), _spec_cnt(), _spec_rows(128), _spec_full(128, 128),
       _spec_full(1, 128), _spec_full(128, 128)],
      _spec_rows(128),
  )(p3, cnt, h2, wl3t, bl3, wr3t)


def _pad_cols(w, n):
  return jnp.concatenate([w, jnp.zeros((w.shape[0], n), jnp.float32)], axis=1)


def _pad_rows(w, n):
  return jnp.concatenate([w, jnp.zeros((n, w.shape[1]), jnp.float32)], axis=0)


@jax.jit
def _run(x, edge_index, Wl0, bl0, Wr0, Wl1, bl1, Wr1, Wl2, bl2, Wr2,
         Wl3, bl3, Wr3):
  xp = jnp.concatenate(
      [x, jnp.zeros((_NP - _N, 128), jnp.float32)], axis=0)
  src80 = edge_index[0]
  dst80 = edge_index[1].reshape(_NWORK, _EPW // 80, 80)
  src125 = edge_index[0].reshape(_NWORK, _EPW // 125, 125)
  dst125 = edge_index[1].reshape(_NWORK, _EPW // 125, 125)

  agg128c = _make_agg(128, 80, 2, True, async_s=False)
  agg64 = _make_agg(64, 125, 4, False)
  agg128 = _make_agg(128, 80, 2, False, async_s=False)

  p0, cnt = agg128c(xp, src80, dst80)
  cnt = cnt.reshape(2, _NP)
  h0, t1 = _stage_a(p0, cnt, xp, Wl0.T, bl0.reshape(1, 128), Wr0.T, Wl1.T)
  (p1,) = agg64(t1, src125, dst125)
  h1 = _stage_b(p1, cnt, h0, bl1.reshape(1, 64), Wr1.T)
  (p2,) = agg64(h1, src125, dst125)
  h2 = _stage_c(p2, cnt, h1, Wl2.T, bl2.reshape(1, 128), Wr2.T)
  (p3,) = agg128(h2, src80, dst80)
  out = _stage_d(p3, cnt, h2, Wl3.T, bl3.reshape(1, 128), Wr3.T)
  return out[:_N]


def kernel(x, edge_index, Wl0, bl0, Wr0, Wl1, bl1, Wr1, Wl2, bl2, Wr2,
           Wl3, bl3, Wr3):
  return _run(x, edge_index, Wl0, bl0, Wr0, Wl1, bl1, Wr1, Wl2, bl2, Wr2,
              Wl3, bl3, Wr3)


# speedup vs baseline: 1.1966x; 1.1966x over previous
import functools

import jax
import jax.numpy as jnp
from jax import lax
from jax.experimental import pallas as pl
from jax.experimental.pallas import tpu as pltpu
from jax.experimental.pallas import tpu_sc as plsc

_N = 10000
_E = 320000
_NP = 10240
_CHUNK = 80
_NWORK = 32
_EPW = _E // _NWORK
_NCH = _EPW // _CHUNK
_ROWS_PT = _NP // 16


def _make_agg(d, chunk, nbuf, with_counts, async_s=True):
  nch = _EPW // chunk
  rem = _EPW - nch * chunk
  assert rem == 0
  mesh = plsc.VectorSubcoreMesh(core_axis_name="c", subcore_axis_name="s")
  src_1d = chunk % 8 == 0
  out_type = [jax.ShapeDtypeStruct((2, _NP, d), jnp.float32)]
  scratch = [
      pltpu.VMEM((_EPW,), jnp.int32) if src_1d
      else pltpu.VMEM((nch, chunk), jnp.int32),
      pltpu.VMEM((nch, chunk), jnp.int32),
  ]
  scratch += [pltpu.VMEM((chunk, d), jnp.float32) for _ in range(nbuf)]
  scratch += [pltpu.SemaphoreType.DMA for _ in range(2 * nbuf)]
  if with_counts:
    out_type.append(jax.ShapeDtypeStruct((2 * _NP,), jnp.float32))
    scratch += [
        pltpu.VMEM((chunk,), jnp.float32),
        pltpu.VMEM((_ROWS_PT,), jnp.float32),
        pltpu.VMEM_SHARED((_NP,), jnp.float32),
    ]
  scratch.append(pltpu.VMEM_SHARED((_NP, d), jnp.float32))

  def body(x_hbm, src_hbm, dst_hbm, out_hbm, *rest):
    if with_counts:
      cnt_hbm = rest[0]
      rest = rest[1:]
    src_v, dst_v = rest[0], rest[1]
    rows = rest[2:2 + nbuf]
    gsem = rest[2 + nbuf:2 + 2 * nbuf]
    ssem = rest[2 + 2 * nbuf:2 + 3 * nbuf]
    rest = rest[2 + 3 * nbuf:]
    if with_counts:
      ones_v, zcnt_v, cnt_acc = rest[0], rest[1], rest[2]
      rest = rest[3:]
    acc = rest[0]
    cid = lax.axis_index("c")
    sid = lax.axis_index("s")
    wid = sid * 2 + cid
    tid = sid

    if src_1d:
      pltpu.sync_copy(src_hbm.at[pl.ds(wid * _EPW, _EPW)], src_v)
    else:
      pltpu.sync_copy(src_hbm.at[wid], src_v)
    pltpu.sync_copy(dst_hbm.at[wid], dst_v)

    def zrow(r, c):
      for cc in range(d // 16):
        rows[0][r, pl.ds(cc * 16, 16)] = jnp.zeros((16,), jnp.float32)
      return c
    lax.fori_loop(0, chunk, zrow, 0)
    nz = _ROWS_PT // chunk
    for j in range(nz):
      pltpu.sync_copy(
          rows[0], acc.at[pl.ds(tid * _ROWS_PT + j * chunk, chunk)])
    zr = _ROWS_PT - nz * chunk
    if zr:
      pltpu.sync_copy(
          rows[0].at[pl.ds(0, zr)],
          acc.at[pl.ds(tid * _ROWS_PT + nz * chunk, zr)])
    if with_counts:
      def zc(i, c):
        zcnt_v[pl.ds(i * 16, 16)] = jnp.zeros((16,), jnp.float32)
        return c
      lax.fori_loop(0, _ROWS_PT // 16, zc, 0)
      pltpu.sync_copy(zcnt_v, cnt_acc.at[pl.ds(tid * _ROWS_PT, _ROWS_PT)])
      for i in range(chunk // 16):
        ones_v[pl.ds(i * 16, 16)] = jnp.ones((16,), jnp.float32)

    def fire_g(k, q):
      idx = (src_v.at[pl.ds(k * chunk, chunk)] if src_1d else src_v.at[k])
      pltpu.async_copy(x_hbm.at[idx], rows[q], gsem[q])

    def wait_g(q):
      pltpu.make_async_copy(
          x_hbm.at[pl.ds(0, chunk)], rows[q], gsem[q]).wait()

    def fire_s(k, q):
      if async_s:
        pltpu.async_copy(rows[q], acc.at[dst_v.at[k]], ssem[q], add=True)
      else:
        pltpu.sync_copy(rows[q], acc.at[dst_v.at[k]], add=True)
      if with_counts:
        pltpu.sync_copy(ones_v, cnt_acc.at[dst_v.at[k]], add=True)

    def wait_s(q):
      if async_s:
        pltpu.make_async_copy(
            x_hbm.at[pl.ds(0, chunk)], rows[q], ssem[q]).wait()

    for q in range(nbuf):
      fire_g(q, q)
    plsc.subcore_barrier()

    full = nch // nbuf
    def grp(kg, c):
      c0 = kg * nbuf
      if async_s:
        for q in range(nbuf):
          wait_g(q)
          fire_s(c0 + q, q)
        for q in range(nbuf):
          nxt = c0 + nbuf + q
          @pl.when(nxt < nch)
          def _():
            wait_s(q)
            fire_g(nxt, q)
      else:
        for q in range(nbuf):
          wait_g(q)
          fire_s(c0 + q, q)
          nxt = c0 + nbuf + q
          @pl.when(nxt < nch)
          def _():
            fire_g(nxt, q)
      return c
    lax.fori_loop(0, full, grp, 0)
    for q in range(nch - full * nbuf):
      wait_g(q)
      fire_s(full * nbuf + q, q)
    for q in range(nbuf):
      wait_s(q)

    plsc.subcore_barrier()
    row0 = tid * _ROWS_PT
    pltpu.sync_copy(acc.at[pl.ds(row0, _ROWS_PT)],
                    out_hbm.at[cid, pl.ds(row0, _ROWS_PT)])
    if with_counts:
      pltpu.sync_copy(cnt_acc.at[pl.ds(row0, _ROWS_PT)],
                      cnt_hbm.at[pl.ds(cid * _NP + row0, _ROWS_PT)])

  params = pltpu.CompilerParams(use_tc_tiling_on_sc=False) if d == 64 else None
  return pl.kernel(body, out_type=tuple(out_type), mesh=mesh,
                   scratch_types=tuple(scratch), compiler_params=params)


_BN = 2048


def _inv_of(cnt_blk):
  c = cnt_blk[0] + cnt_blk[1]
  return (1.0 / jnp.maximum(c, 1.0))[:, None]


def _dot(a, b):
  return jax.lax.dot_general(a, b, (((1,), (0,)), ((), ())),
                             preferred_element_type=jnp.float32)


def _tc_call(body, out_shapes, in_specs, out_specs):
  return pl.pallas_call(
      body,
      grid=(_NP // _BN,),
      in_specs=in_specs,
      out_specs=out_specs,
      out_shape=out_shapes,
  )


def _spec_rows(d):
  return pl.BlockSpec((_BN, d), lambda i: (i, 0))


def _spec_pair(d):
  return pl.BlockSpec((2, _BN, d), lambda i: (0, i, 0))


def _spec_cnt():
  return pl.BlockSpec((2, _BN), lambda i: (0, i))


def _spec_full(r, c):
  return pl.BlockSpec((r, c), lambda i: (0, 0))


def _stage_a(p0, cnt, x, wl0t, bl0, wr0t, wl1t):
  def body(p_ref, c_ref, x_ref, wl_ref, b_ref, wr_ref, w1_ref, h0_ref, t1_ref):
    m = (p_ref[0] + p_ref[1]) * _inv_of(c_ref)
    h0 = jnp.maximum(
        _dot(m, wl_ref[...]) + b_ref[...] + _dot(x_ref[...], wr_ref[...]), 0.0)
    h0_ref[...] = h0
    t1_ref[...] = _dot(h0, w1_ref[...])
  return _tc_call(
      body,
      (jax.ShapeDtypeStruct((_NP, 128), jnp.float32),
       jax.ShapeDtypeStruct((_NP, 64), jnp.float32)),
      [_spec_pair(128), _spec_cnt(), _spec_rows(128), _spec_full(128, 128),
       _spec_full(1, 128), _spec_full(128, 128), _spec_full(128, 64)],
      (_spec_rows(128), _spec_rows(64)),
  )(p0, cnt, x, wl0t, bl0, wr0t, wl1t)


def _stage_b(p1, cnt, h0, bl1, wr1t):
  def body(p_ref, c_ref, h_ref, b_ref, wr_ref, o_ref):
    m = (p_ref[0] + p_ref[1]) * _inv_of(c_ref)
    o_ref[...] = m + b_ref[...] + _dot(h_ref[...], wr_ref[...])
  return _tc_call(
      body,
      jax.ShapeDtypeStruct((_NP, 64), jnp.float32),
      [_spec_pair(64), _spec_cnt(), _spec_rows(128), _spec_full(1, 64),
       _spec_full(128, 64)],
      _spec_rows(64),
  )(p1, cnt, h0, bl1, wr1t)


def _stage_c(p2, cnt, h1, wl2t, bl2, wr2t):
  def body(p_ref, c_ref, h_ref, wl_ref, b_ref, wr_ref, o_ref):
    m = (p_ref[0] + p_ref[1]) * _inv_of(c_ref)
    o_ref[...] = jnp.maximum(
        _dot(m, wl_ref[...]) + b_ref[...] + _dot(h_ref[...], wr_ref[...]), 0.0)
  return _tc_call(
      body,
      jax.ShapeDtypeStruct((_NP, 128), jnp.float32),
      [_spec_pair(64), _spec_cnt(), _spec_rows(64), _spec_full(64, 128),
       _spec_full(1, 128), _spec_full(64, 128)],
      _spec_rows(128),
  )(p2, cnt, h1, wl2t, bl2, wr2t)


def _stage_d(p3, cnt, h2, wl3t, bl3, wr3t):
  def body(p_ref, c_ref, h_ref, wl_ref, b_ref, wr_ref, o_ref):
    m = (p_ref[0] + p_ref[1]) * _inv_of(c_ref)
    o_ref[...] = (_dot(m, wl_ref[...]) + b_ref[...]
                  + _dot(h_ref[...], wr_ref[...]))
  return _tc_call(
      body,
      jax.ShapeDtypeStruct((_NP, 128), jnp.float32),
      [_spec_pair(128), _spec_cnt(), _spec_rows(128), _spec_full(128, 128),
       _spec_full(1, 128), _spec_full(128, 128)],
      _spec_rows(128),
  )(p3, cnt, h2, wl3t, bl3, wr3t)


def _pad_cols(w, n):
  return jnp.concatenate([w, jnp.zeros((w.shape[0], n), jnp.float32)], axis=1)


def _pad_rows(w, n):
  return jnp.concatenate([w, jnp.zeros((n, w.shape[1]), jnp.float32)], axis=0)


@jax.jit
def _run(x, edge_index, Wl0, bl0, Wr0, Wl1, bl1, Wr1, Wl2, bl2, Wr2,
         Wl3, bl3, Wr3):
  xp = jnp.concatenate(
      [x, jnp.zeros((_NP - _N, 128), jnp.float32)], axis=0)
  src80 = edge_index[0]
  dst80 = edge_index[1].reshape(_NWORK, _EPW // 80, 80)
  src125 = edge_index[0].reshape(_NWORK, _EPW // 125, 125)
  dst125 = edge_index[1].reshape(_NWORK, _EPW // 125, 125)

  agg128c = _make_agg(128, 80, 2, True, async_s=False)
  agg64 = _make_agg(64, 125, 4, False)
  agg128 = _make_agg(128, 80, 2, False, async_s=False)

  p0, cnt = agg128c(xp, src80, dst80)
  cnt = cnt.reshape(2, _NP)
  h0, t1 = _stage_a(p0, cnt, xp, Wl0.T, bl0.reshape(1, 128), Wr0.T, Wl1.T)
  (p1,) = agg64(t1, src125, dst125)
  h1 = _stage_b(p1, cnt, h0, bl1.reshape(1, 64), Wr1.T)
  (p2,) = agg64(h1, src125, dst125)
  h2 = _stage_c(p2, cnt, h1, Wl2.T, bl2.reshape(1, 128), Wr2.T)
  (p3,) = agg128(h2, src80, dst80)
  out = _stage_d(p3, cnt, h2, Wl3.T, bl3.reshape(1, 128), Wr3.T)
  return out[:_N]


def kernel(x, edge_index, Wl0, bl0, Wr0, Wl1, bl1, Wr1, Wl2, bl2, Wr2,
           Wl3, bl3, Wr3):
  return _run(x, edge_index, Wl0, bl0, Wr0, Wl1, bl1, Wr1, Wl2, bl2, Wr2,
              Wl3, bl3, Wr3)


# speedup vs baseline: 1.2215x; 1.0208x over previous
import functools

import jax
import jax.numpy as jnp
from jax import lax
from jax.experimental import pallas as pl
from jax.experimental.pallas import tpu as pltpu
from jax.experimental.pallas import tpu_sc as plsc

_N = 10000
_E = 320000
_NP = 10240
_CHUNK = 80
_NWORK = 32
_EPW = _E // _NWORK
_NCH = _EPW // _CHUNK
_ROWS_PT = _NP // 16


def _make_agg(d, chunk, nbuf, with_counts, async_s=True):
  nch = _EPW // chunk
  rem = _EPW - nch * chunk
  assert rem == 0
  mesh = plsc.VectorSubcoreMesh(core_axis_name="c", subcore_axis_name="s")
  src_1d = chunk % 8 == 0
  out_type = [jax.ShapeDtypeStruct((2, _NP, d), jnp.float32)]
  scratch = [
      pltpu.VMEM((_EPW,), jnp.int32) if src_1d
      else pltpu.VMEM((nch, chunk), jnp.int32),
      pltpu.VMEM((nch, chunk), jnp.int32),
  ]
  scratch += [pltpu.VMEM((chunk, d), jnp.float32) for _ in range(nbuf)]
  scratch += [pltpu.SemaphoreType.DMA for _ in range(2 * nbuf)]
  if with_counts:
    out_type.append(jax.ShapeDtypeStruct((2 * _NP,), jnp.float32))
    scratch += [
        pltpu.VMEM((chunk,), jnp.float32),
        pltpu.VMEM((_ROWS_PT,), jnp.float32),
        pltpu.VMEM_SHARED((_NP,), jnp.float32),
    ]
  scratch.append(pltpu.VMEM_SHARED((_NP, d), jnp.float32))

  def body(x_hbm, src_hbm, dst_hbm, out_hbm, *rest):
    if with_counts:
      cnt_hbm = rest[0]
      rest = rest[1:]
    src_v, dst_v = rest[0], rest[1]
    rows = rest[2:2 + nbuf]
    gsem = rest[2 + nbuf:2 + 2 * nbuf]
    ssem = rest[2 + 2 * nbuf:2 + 3 * nbuf]
    rest = rest[2 + 3 * nbuf:]
    if with_counts:
      ones_v, zcnt_v, cnt_acc = rest[0], rest[1], rest[2]
      rest = rest[3:]
    acc = rest[0]
    cid = lax.axis_index("c")
    sid = lax.axis_index("s")
    wid = sid * 2 + cid
    tid = sid

    if src_1d:
      pltpu.async_copy(src_hbm.at[pl.ds(wid * _EPW, _EPW)], src_v, gsem[0])
    else:
      pltpu.async_copy(src_hbm.at[wid], src_v, gsem[0])
    pltpu.async_copy(dst_hbm.at[wid], dst_v, gsem[0])

    def zrow(r, c):
      for cc in range(d // 16):
        rows[0][r, pl.ds(cc * 16, 16)] = jnp.zeros((16,), jnp.float32)
      return c
    lax.fori_loop(0, chunk, zrow, 0)
    nz = _ROWS_PT // chunk
    zr = _ROWS_PT - nz * chunk
    for j in range(nz):
      pltpu.async_copy(
          rows[0], acc.at[pl.ds(tid * _ROWS_PT + j * chunk, chunk)], ssem[0])
    if zr:
      pltpu.async_copy(
          rows[0].at[pl.ds(0, zr)],
          acc.at[pl.ds(tid * _ROWS_PT + nz * chunk, zr)], ssem[0])
    pltpu.make_async_copy(src_hbm.at[wid] if not src_1d
                          else src_hbm.at[pl.ds(0, _EPW)], src_v,
                          gsem[0]).wait()
    pltpu.make_async_copy(dst_hbm.at[0], dst_v, gsem[0]).wait()
    for j in range(nz):
      pltpu.make_async_copy(
          rows[0], acc.at[pl.ds(tid * _ROWS_PT + j * chunk, chunk)],
          ssem[0]).wait()
    if zr:
      pltpu.make_async_copy(
          rows[0].at[pl.ds(0, zr)],
          acc.at[pl.ds(tid * _ROWS_PT + nz * chunk, zr)], ssem[0]).wait()
    if with_counts:
      def zc(i, c):
        zcnt_v[pl.ds(i * 16, 16)] = jnp.zeros((16,), jnp.float32)
        return c
      lax.fori_loop(0, _ROWS_PT // 16, zc, 0)
      pltpu.sync_copy(zcnt_v, cnt_acc.at[pl.ds(tid * _ROWS_PT, _ROWS_PT)])
      for i in range(chunk // 16):
        ones_v[pl.ds(i * 16, 16)] = jnp.ones((16,), jnp.float32)

    def fire_g(k, q):
      idx = (src_v.at[pl.ds(k * chunk, chunk)] if src_1d else src_v.at[k])
      pltpu.async_copy(x_hbm.at[idx], rows[q], gsem[q])

    def wait_g(q):
      pltpu.make_async_copy(
          x_hbm.at[pl.ds(0, chunk)], rows[q], gsem[q]).wait()

    def fire_s(k, q):
      if async_s:
        pltpu.async_copy(rows[q], acc.at[dst_v.at[k]], ssem[q], add=True)
      else:
        pltpu.sync_copy(rows[q], acc.at[dst_v.at[k]], add=True)
      if with_counts:
        pltpu.sync_copy(ones_v, cnt_acc.at[dst_v.at[k]], add=True)

    def wait_s(q):
      if async_s:
        pltpu.make_async_copy(
            x_hbm.at[pl.ds(0, chunk)], rows[q], ssem[q]).wait()

    for q in range(nbuf):
      fire_g(q, q)
    plsc.subcore_barrier()

    full = nch // nbuf
    def grp(kg, c):
      c0 = kg * nbuf
      if async_s:
        for q in range(nbuf):
          wait_g(q)
          fire_s(c0 + q, q)
        for q in range(nbuf):
          nxt = c0 + nbuf + q
          @pl.when(nxt < nch)
          def _():
            wait_s(q)
            fire_g(nxt, q)
      else:
        for q in range(nbuf):
          wait_g(q)
          fire_s(c0 + q, q)
          nxt = c0 + nbuf + q
          @pl.when(nxt < nch)
          def _():
            fire_g(nxt, q)
      return c
    lax.fori_loop(0, full, grp, 0)
    for q in range(nch - full * nbuf):
      wait_g(q)
      fire_s(full * nbuf + q, q)
    for q in range(nbuf):
      wait_s(q)

    plsc.subcore_barrier()
    row0 = tid * _ROWS_PT
    pltpu.async_copy(acc.at[pl.ds(row0, _ROWS_PT)],
                     out_hbm.at[cid, pl.ds(row0, _ROWS_PT)], gsem[0])
    if with_counts:
      pltpu.async_copy(cnt_acc.at[pl.ds(row0, _ROWS_PT)],
                       cnt_hbm.at[pl.ds(cid * _NP + row0, _ROWS_PT)], gsem[0])
    pltpu.make_async_copy(acc.at[pl.ds(row0, _ROWS_PT)],
                          out_hbm.at[cid, pl.ds(row0, _ROWS_PT)],
                          gsem[0]).wait()
    if with_counts:
      pltpu.make_async_copy(cnt_acc.at[pl.ds(row0, _ROWS_PT)],
                            cnt_hbm.at[pl.ds(cid * _NP + row0, _ROWS_PT)],
                            gsem[0]).wait()

  params = pltpu.CompilerParams(use_tc_tiling_on_sc=False) if d == 64 else None
  return pl.kernel(body, out_type=tuple(out_type), mesh=mesh,
                   scratch_types=tuple(scratch), compiler_params=params)


_BN = 2048


def _inv_of(cnt_blk):
  c = cnt_blk[0] + cnt_blk[1]
  return (1.0 / jnp.maximum(c, 1.0))[:, None]


def _dot(a, b):
  return jax.lax.dot_general(a, b, (((1,), (0,)), ((), ())),
                             preferred_element_type=jnp.float32)


def _tc_call(body, out_shapes, in_specs, out_specs):
  return pl.pallas_call(
      body,
      grid=(_NP // _BN,),
      in_specs=in_specs,
      out_specs=out_specs,
      out_shape=out_shapes,
  )


def _spec_rows(d):
  return pl.BlockSpec((_BN, d), lambda i: (i, 0))


def _spec_pair(d):
  return pl.BlockSpec((2, _BN, d), lambda i: (0, i, 0))


def _spec_cnt():
  return pl.BlockSpec((2, _BN), lambda i: (0, i))


def _spec_full(r, c):
  return pl.BlockSpec((r, c), lambda i: (0, 0))


def _stage_a(p0, cnt, x, wl0t, bl0, wr0t, wl1t):
  def body(p_ref, c_ref, x_ref, wl_ref, b_ref, wr_ref, w1_ref, h0_ref, t1_ref):
    m = (p_ref[0] + p_ref[1]) * _inv_of(c_ref)
    h0 = jnp.maximum(
        _dot(m, wl_ref[...]) + b_ref[...] + _dot(x_ref[...], wr_ref[...]), 0.0)
    h0_ref[...] = h0
    t1_ref[...] = _dot(h0, w1_ref[...])
  return _tc_call(
      body,
      (jax.ShapeDtypeStruct((_NP, 128), jnp.float32),
       jax.ShapeDtypeStruct((_NP, 64), jnp.float32)),
      [_spec_pair(128), _spec_cnt(), _spec_rows(128), _spec_full(128, 128),
       _spec_full(1, 128), _spec_full(128, 128), _spec_full(128, 64)],
      (_spec_rows(128), _spec_rows(64)),
  )(p0, cnt, x, wl0t, bl0, wr0t, wl1t)


def _stage_b(p1, cnt, h0, bl1, wr1t):
  def body(p_ref, c_ref, h_ref, b_ref, wr_ref, o_ref):
    m = (p_ref[0] + p_ref[1]) * _inv_of(c_ref)
    o_ref[...] = m + b_ref[...] + _dot(h_ref[...], wr_ref[...])
  return _tc_call(
      body,
      jax.ShapeDtypeStruct((_NP, 64), jnp.float32),
      [_spec_pair(64), _spec_cnt(), _spec_rows(128), _spec_full(1, 64),
       _spec_full(128, 64)],
      _spec_rows(64),
  )(p1, cnt, h0, bl1, wr1t)


def _stage_c(p2, cnt, h1, wl2t, bl2, wr2t):
  def body(p_ref, c_ref, h_ref, wl_ref, b_ref, wr_ref, o_ref):
    m = (p_ref[0] + p_ref[1]) * _inv_of(c_ref)
    o_ref[...] = jnp.maximum(
        _dot(m, wl_ref[...]) + b_ref[...] + _dot(h_ref[...], wr_ref[...]), 0.0)
  return _tc_call(
      body,
      jax.ShapeDtypeStruct((_NP, 128), jnp.float32),
      [_spec_pair(64), _spec_cnt(), _spec_rows(64), _spec_full(64, 128),
       _spec_full(1, 128), _spec_full(64, 128)],
      _spec_rows(128),
  )(p2, cnt, h1, wl2t, bl2, wr2t)


def _stage_d(p3, cnt, h2, wl3t, bl3, wr3t):
  def body(p_ref, c_ref, h_ref, wl_ref, b_ref, wr_ref, o_ref):
    m = (p_ref[0] + p_ref[1]) * _inv_of(c_ref)
    o_ref[...] = (_dot(m, wl_ref[...]) + b_ref[...]
                  + _dot(h_ref[...], wr_ref[...]))
  return _tc_call(
      body,
      jax.ShapeDtypeStruct((_NP, 128), jnp.float32),
      [_spec_pair(128), _spec_cnt(), _spec_rows(128), _spec_full(128, 128),
       _spec_full(1, 128), _spec_full(128, 128)],
      _spec_rows(128),
  )(p3, cnt, h2, wl3t, bl3, wr3t)


def _pad_cols(w, n):
  return jnp.concatenate([w, jnp.zeros((w.shape[0], n), jnp.float32)], axis=1)


def _pad_rows(w, n):
  return jnp.concatenate([w, jnp.zeros((n, w.shape[1]), jnp.float32)], axis=0)


@jax.jit
def _run(x, edge_index, Wl0, bl0, Wr0, Wl1, bl1, Wr1, Wl2, bl2, Wr2,
         Wl3, bl3, Wr3):
  xp = jnp.concatenate(
      [x, jnp.zeros((_NP - _N, 128), jnp.float32)], axis=0)
  src80 = edge_index[0]
  dst80 = edge_index[1].reshape(_NWORK, _EPW // 80, 80)
  src125 = edge_index[0].reshape(_NWORK, _EPW // 125, 125)
  dst125 = edge_index[1].reshape(_NWORK, _EPW // 125, 125)

  agg128c = _make_agg(128, 80, 2, True, async_s=False)
  agg64 = _make_agg(64, 125, 4, False)
  agg128 = _make_agg(128, 80, 2, False, async_s=False)

  p0, cnt = agg128c(xp, src80, dst80)
  cnt = cnt.reshape(2, _NP)
  h0, t1 = _stage_a(p0, cnt, xp, Wl0.T, bl0.reshape(1, 128), Wr0.T, Wl1.T)
  (p1,) = agg64(t1, src125, dst125)
  h1 = _stage_b(p1, cnt, h0, bl1.reshape(1, 64), Wr1.T)
  (p2,) = agg64(h1, src125, dst125)
  h2 = _stage_c(p2, cnt, h1, Wl2.T, bl2.reshape(1, 128), Wr2.T)
  (p3,) = agg128(h2, src80, dst80)
  out = _stage_d(p3, cnt, h2, Wl3.T, bl3.reshape(1, 128), Wr3.T)
  return out[:_N]


def kernel(x, edge_index, Wl0, bl0, Wr0, Wl1, bl1, Wr1, Wl2, bl2, Wr2,
           Wl3, bl3, Wr3):
  return _run(x, edge_index, Wl0, bl0, Wr0, Wl1, bl1, Wr1, Wl2, bl2, Wr2,
              Wl3, bl3, Wr3)


# speedup vs baseline: 1.2522x; 1.0252x over previous
import functools

import jax
import jax.numpy as jnp
from jax import lax
from jax.experimental import pallas as pl
from jax.experimental.pallas import tpu as pltpu
from jax.experimental.pallas import tpu_sc as plsc

_N = 10000
_E = 320000
_NP = 10240
_CHUNK = 80
_NWORK = 32
_EPW = _E // _NWORK
_NCH = _EPW // _CHUNK
_ROWS_PT = _NP // 16


def _make_agg(d, chunk, nbuf, with_counts, async_s=True,
              dt=jnp.float32):
  nch = _EPW // chunk
  rem = _EPW - nch * chunk
  assert rem == 0
  mesh = plsc.VectorSubcoreMesh(core_axis_name="c", subcore_axis_name="s")
  src_1d = chunk % 8 == 0
  out_type = [jax.ShapeDtypeStruct((2, _NP, d), dt)]
  scratch = [
      pltpu.VMEM((_EPW,), jnp.int32) if src_1d
      else pltpu.VMEM((nch, chunk), jnp.int32),
      pltpu.VMEM((nch, chunk), jnp.int32),
  ]
  scratch += [pltpu.VMEM((chunk, d), dt) for _ in range(nbuf)]
  scratch += [pltpu.SemaphoreType.DMA for _ in range(2 * nbuf)]
  if with_counts:
    out_type.append(jax.ShapeDtypeStruct((2 * _NP,), jnp.float32))
    scratch += [
        pltpu.VMEM((chunk,), jnp.float32),
        pltpu.VMEM((_ROWS_PT,), jnp.float32),
        pltpu.VMEM_SHARED((_NP,), jnp.float32),
    ]
  scratch.append(pltpu.VMEM_SHARED((_NP, d), dt))

  def body(x_hbm, src_hbm, dst_hbm, out_hbm, *rest):
    if with_counts:
      cnt_hbm = rest[0]
      rest = rest[1:]
    src_v, dst_v = rest[0], rest[1]
    rows = rest[2:2 + nbuf]
    gsem = rest[2 + nbuf:2 + 2 * nbuf]
    ssem = rest[2 + 2 * nbuf:2 + 3 * nbuf]
    rest = rest[2 + 3 * nbuf:]
    if with_counts:
      ones_v, zcnt_v, cnt_acc = rest[0], rest[1], rest[2]
      rest = rest[3:]
    acc = rest[0]
    cid = lax.axis_index("c")
    sid = lax.axis_index("s")
    wid = sid * 2 + cid
    tid = sid

    if src_1d:
      pltpu.async_copy(src_hbm.at[pl.ds(wid * _EPW, _EPW)], src_v, gsem[0])
    else:
      pltpu.async_copy(src_hbm.at[wid], src_v, gsem[0])
    pltpu.async_copy(dst_hbm.at[wid], dst_v, gsem[0])

    lanes = 16 if dt == jnp.float32 else 32
    def zrow(r, c):
      for cc in range(d // lanes):
        rows[0][r, pl.ds(cc * lanes, lanes)] = jnp.zeros((lanes,), dt)
      return c
    lax.fori_loop(0, chunk, zrow, 0)
    nz = _ROWS_PT // chunk
    zr = _ROWS_PT - nz * chunk
    for j in range(nz):
      pltpu.async_copy(
          rows[0], acc.at[pl.ds(tid * _ROWS_PT + j * chunk, chunk)], ssem[0])
    if zr:
      pltpu.async_copy(
          rows[0].at[pl.ds(0, zr)],
          acc.at[pl.ds(tid * _ROWS_PT + nz * chunk, zr)], ssem[0])
    pltpu.make_async_copy(src_hbm.at[wid] if not src_1d
                          else src_hbm.at[pl.ds(0, _EPW)], src_v,
                          gsem[0]).wait()
    pltpu.make_async_copy(dst_hbm.at[0], dst_v, gsem[0]).wait()
    for j in range(nz):
      pltpu.make_async_copy(
          rows[0], acc.at[pl.ds(tid * _ROWS_PT + j * chunk, chunk)],
          ssem[0]).wait()
    if zr:
      pltpu.make_async_copy(
          rows[0].at[pl.ds(0, zr)],
          acc.at[pl.ds(tid * _ROWS_PT + nz * chunk, zr)], ssem[0]).wait()
    if with_counts:
      def zc(i, c):
        zcnt_v[pl.ds(i * 16, 16)] = jnp.zeros((16,), jnp.float32)
        return c
      lax.fori_loop(0, _ROWS_PT // 16, zc, 0)
      pltpu.sync_copy(zcnt_v, cnt_acc.at[pl.ds(tid * _ROWS_PT, _ROWS_PT)])
      for i in range(chunk // 16):
        ones_v[pl.ds(i * 16, 16)] = jnp.ones((16,), jnp.float32)

    def fire_g(k, q):
      idx = (src_v.at[pl.ds(k * chunk, chunk)] if src_1d else src_v.at[k])
      pltpu.async_copy(x_hbm.at[idx], rows[q], gsem[q])

    def wait_g(q):
      pltpu.make_async_copy(
          x_hbm.at[pl.ds(0, chunk)], rows[q], gsem[q]).wait()

    def fire_s(k, q):
      if async_s:
        pltpu.async_copy(rows[q], acc.at[dst_v.at[k]], ssem[q], add=True)
      else:
        pltpu.sync_copy(rows[q], acc.at[dst_v.at[k]], add=True)
      if with_counts:
        pltpu.sync_copy(ones_v, cnt_acc.at[dst_v.at[k]], add=True)

    def wait_s(q):
      if async_s:
        pltpu.make_async_copy(
            x_hbm.at[pl.ds(0, chunk)], rows[q], ssem[q]).wait()

    for q in range(nbuf):
      fire_g(q, q)
    plsc.subcore_barrier()

    full = nch // nbuf
    def grp(kg, c):
      c0 = kg * nbuf
      if async_s:
        for q in range(nbuf):
          wait_g(q)
          fire_s(c0 + q, q)
        for q in range(nbuf):
          nxt = c0 + nbuf + q
          @pl.when(nxt < nch)
          def _():
            wait_s(q)
            fire_g(nxt, q)
      else:
        for q in range(nbuf):
          wait_g(q)
          fire_s(c0 + q, q)
          nxt = c0 + nbuf + q
          @pl.when(nxt < nch)
          def _():
            fire_g(nxt, q)
      return c
    lax.fori_loop(0, full, grp, 0)
    for q in range(nch - full * nbuf):
      wait_g(q)
      fire_s(full * nbuf + q, q)
    for q in range(nbuf):
      wait_s(q)

    plsc.subcore_barrier()
    row0 = tid * _ROWS_PT
    pltpu.async_copy(acc.at[pl.ds(row0, _ROWS_PT)],
                     out_hbm.at[cid, pl.ds(row0, _ROWS_PT)], gsem[0])
    if with_counts:
      pltpu.async_copy(cnt_acc.at[pl.ds(row0, _ROWS_PT)],
                       cnt_hbm.at[pl.ds(cid * _NP + row0, _ROWS_PT)], gsem[0])
    pltpu.make_async_copy(acc.at[pl.ds(row0, _ROWS_PT)],
                          out_hbm.at[cid, pl.ds(row0, _ROWS_PT)],
                          gsem[0]).wait()
    if with_counts:
      pltpu.make_async_copy(cnt_acc.at[pl.ds(row0, _ROWS_PT)],
                            cnt_hbm.at[pl.ds(cid * _NP + row0, _ROWS_PT)],
                            gsem[0]).wait()

  untiled = d == 64 or dt == jnp.bfloat16
  params = pltpu.CompilerParams(use_tc_tiling_on_sc=False) if untiled else None
  return pl.kernel(body, out_type=tuple(out_type), mesh=mesh,
                   scratch_types=tuple(scratch), compiler_params=params)


_BN = 2048


def _inv_of(cnt_blk):
  c = cnt_blk[0] + cnt_blk[1]
  return (1.0 / jnp.maximum(c, 1.0))[:, None]


def _psum(p_ref):
  return (p_ref[0].astype(jnp.float32) + p_ref[1].astype(jnp.float32))


def _dot(a, b):
  return jax.lax.dot_general(a, b, (((1,), (0,)), ((), ())),
                             preferred_element_type=jnp.float32)


def _tc_call(body, out_shapes, in_specs, out_specs):
  return pl.pallas_call(
      body,
      grid=(_NP // _BN,),
      in_specs=in_specs,
      out_specs=out_specs,
      out_shape=out_shapes,
  )


def _spec_rows(d):
  return pl.BlockSpec((_BN, d), lambda i: (i, 0))


def _spec_pair(d):
  return pl.BlockSpec((2, _BN, d), lambda i: (0, i, 0))


def _spec_cnt():
  return pl.BlockSpec((2, _BN), lambda i: (0, i))


def _spec_full(r, c):
  return pl.BlockSpec((r, c), lambda i: (0, 0))


def _stage_a(p0, cnt, x, wl0t, bl0, wr0t, wl1t):
  def body(p_ref, c_ref, x_ref, wl_ref, b_ref, wr_ref, w1_ref, h0_ref, t1_ref):
    m = _psum(p_ref) * _inv_of(c_ref)
    h0 = jnp.maximum(
        _dot(m, wl_ref[...]) + b_ref[...] + _dot(x_ref[...], wr_ref[...]), 0.0)
    h0_ref[...] = h0
    t1_ref[...] = _dot(h0, w1_ref[...])
  return _tc_call(
      body,
      (jax.ShapeDtypeStruct((_NP, 128), jnp.float32),
       jax.ShapeDtypeStruct((_NP, 64), jnp.float32)),
      [_spec_pair(128), _spec_cnt(), _spec_rows(128), _spec_full(128, 128),
       _spec_full(1, 128), _spec_full(128, 128), _spec_full(128, 64)],
      (_spec_rows(128), _spec_rows(64)),
  )(p0, cnt, x, wl0t, bl0, wr0t, wl1t)


def _stage_b(p1, cnt, h0, bl1, wr1t):
  def body(p_ref, c_ref, h_ref, b_ref, wr_ref, o_ref):
    m = _psum(p_ref) * _inv_of(c_ref)
    o_ref[...] = m + b_ref[...] + _dot(h_ref[...], wr_ref[...])
  return _tc_call(
      body,
      jax.ShapeDtypeStruct((_NP, 64), jnp.float32),
      [_spec_pair(64), _spec_cnt(), _spec_rows(128), _spec_full(1, 64),
       _spec_full(128, 64)],
      _spec_rows(64),
  )(p1, cnt, h0, bl1, wr1t)


def _stage_c(p2, cnt, h1, wl2t, bl2, wr2t):
  def body(p_ref, c_ref, h_ref, wl_ref, b_ref, wr_ref, o_ref, ob_ref):
    m = _psum(p_ref) * _inv_of(c_ref)
    h2 = jnp.maximum(
        _dot(m, wl_ref[...]) + b_ref[...] + _dot(h_ref[...], wr_ref[...]), 0.0)
    o_ref[...] = h2
    ob_ref[...] = h2.astype(jnp.bfloat16)
  return _tc_call(
      body,
      (jax.ShapeDtypeStruct((_NP, 128), jnp.float32),
       jax.ShapeDtypeStruct((_NP, 128), jnp.bfloat16)),
      [_spec_pair(64), _spec_cnt(), _spec_rows(64), _spec_full(64, 128),
       _spec_full(1, 128), _spec_full(64, 128)],
      (_spec_rows(128), _spec_rows(128)),
  )(p2, cnt, h1, wl2t, bl2, wr2t)


def _stage_d(p3, cnt, h2, wl3t, bl3, wr3t):
  def body(p_ref, c_ref, h_ref, wl_ref, b_ref, wr_ref, o_ref):
    m = _psum(p_ref) * _inv_of(c_ref)
    o_ref[...] = (_dot(m, wl_ref[...]) + b_ref[...]
                  + _dot(h_ref[...], wr_ref[...]))
  return _tc_call(
      body,
      jax.ShapeDtypeStruct((_NP, 128), jnp.float32),
      [_spec_pair(128), _spec_cnt(), _spec_rows(128), _spec_full(128, 128),
       _spec_full(1, 128), _spec_full(128, 128)],
      _spec_rows(128),
  )(p3, cnt, h2, wl3t, bl3, wr3t)


def _pad_cols(w, n):
  return jnp.concatenate([w, jnp.zeros((w.shape[0], n), jnp.float32)], axis=1)


def _pad_rows(w, n):
  return jnp.concatenate([w, jnp.zeros((n, w.shape[1]), jnp.float32)], axis=0)


@jax.jit
def _run(x, edge_index, Wl0, bl0, Wr0, Wl1, bl1, Wr1, Wl2, bl2, Wr2,
         Wl3, bl3, Wr3):
  xp = jnp.concatenate(
      [x, jnp.zeros((_NP - _N, 128), jnp.float32)], axis=0)
  src80 = edge_index[0]
  dst80 = edge_index[1].reshape(_NWORK, _EPW // 80, 80)
  src125 = edge_index[0].reshape(_NWORK, _EPW // 125, 125)
  dst125 = edge_index[1].reshape(_NWORK, _EPW // 125, 125)

  agg128c = _make_agg(128, 80, 2, True, async_s=False, dt=jnp.bfloat16)
  agg64 = _make_agg(64, 125, 4, False)
  agg128 = _make_agg(128, 80, 2, False, async_s=False, dt=jnp.bfloat16)

  xbf = xp.astype(jnp.bfloat16)
  p0, cnt = agg128c(xbf, src80, dst80)
  cnt = cnt.reshape(2, _NP)
  h0, t1 = _stage_a(p0, cnt, xp, Wl0.T, bl0.reshape(1, 128), Wr0.T, Wl1.T)
  (p1,) = agg64(t1, src125, dst125)
  h1 = _stage_b(p1, cnt, h0, bl1.reshape(1, 64), Wr1.T)
  (p2,) = agg64(h1, src125, dst125)
  h2, h2bf = _stage_c(p2, cnt, h1, Wl2.T, bl2.reshape(1, 128), Wr2.T)
  (p3,) = agg128(h2bf, src80, dst80)
  out = _stage_d(p3, cnt, h2, Wl3.T, bl3.reshape(1, 128), Wr3.T)
  return out[:_N]


def kernel(x, edge_index, Wl0, bl0, Wr0, Wl1, bl1, Wr1, Wl2, bl2, Wr2,
           Wl3, bl3, Wr3):
  return _run(x, edge_index, Wl0, bl0, Wr0, Wl1, bl1, Wr1, Wl2, bl2, Wr2,
              Wl3, bl3, Wr3)


# speedup vs baseline: 1.2951x; 1.0343x over previous
import functools

import jax
import jax.numpy as jnp
from jax import lax
from jax.experimental import pallas as pl
from jax.experimental.pallas import tpu as pltpu
from jax.experimental.pallas import tpu_sc as plsc

_N = 10000
_E = 320000
_NP = 10240
_CHUNK = 80
_NWORK = 32
_EPW = _E // _NWORK
_NCH = _EPW // _CHUNK
_ROWS_PT = _NP // 16


def _make_agg(d, chunk, nbuf, with_counts, async_s=True,
              dt=jnp.float32):
  nch = _EPW // chunk
  rem = _EPW - nch * chunk
  assert rem == 0
  mesh = plsc.VectorSubcoreMesh(core_axis_name="c", subcore_axis_name="s")
  src_1d = chunk % 8 == 0
  out_type = [jax.ShapeDtypeStruct((2, _NP, d), dt)]
  scratch = [
      pltpu.VMEM((_EPW,), jnp.int32) if src_1d
      else pltpu.VMEM((nch, chunk), jnp.int32),
      pltpu.VMEM((nch, chunk), jnp.int32),
  ]
  scratch += [pltpu.VMEM((chunk, d), dt) for _ in range(nbuf)]
  scratch += [pltpu.SemaphoreType.DMA for _ in range(2 * nbuf)]
  if with_counts:
    out_type.append(jax.ShapeDtypeStruct((2 * _NP,), jnp.float32))
    scratch += [
        pltpu.VMEM((128,), jnp.float32),
        pltpu.VMEM((_ROWS_PT,), jnp.float32),
        pltpu.VMEM_SHARED((_NP,), jnp.float32),
    ]
  scratch.append(pltpu.VMEM_SHARED((_NP, d), dt))

  def body(x_hbm, src_hbm, dst_hbm, out_hbm, *rest):
    if with_counts:
      cnt_hbm = rest[0]
      rest = rest[1:]
    src_v, dst_v = rest[0], rest[1]
    rows = rest[2:2 + nbuf]
    gsem = rest[2 + nbuf:2 + 2 * nbuf]
    ssem = rest[2 + 2 * nbuf:2 + 3 * nbuf]
    rest = rest[2 + 3 * nbuf:]
    if with_counts:
      ones_v, zcnt_v, cnt_acc = rest[0], rest[1], rest[2]
      rest = rest[3:]
    acc = rest[0]
    cid = lax.axis_index("c")
    sid = lax.axis_index("s")
    wid = sid * 2 + cid
    tid = sid

    if src_1d:
      pltpu.async_copy(src_hbm.at[pl.ds(wid * _EPW, _EPW)], src_v, gsem[0])
    else:
      pltpu.async_copy(src_hbm.at[wid], src_v, gsem[0])
    pltpu.async_copy(dst_hbm.at[wid], dst_v, gsem[0])

    lanes = 16 if dt == jnp.float32 else 32
    def zrow(r, c):
      for cc in range(d // lanes):
        rows[0][r, pl.ds(cc * lanes, lanes)] = jnp.zeros((lanes,), dt)
      return c
    lax.fori_loop(0, chunk, zrow, 0)
    nz = _ROWS_PT // chunk
    zr = _ROWS_PT - nz * chunk
    for j in range(nz):
      pltpu.async_copy(
          rows[0], acc.at[pl.ds(tid * _ROWS_PT + j * chunk, chunk)], ssem[0])
    if zr:
      pltpu.async_copy(
          rows[0].at[pl.ds(0, zr)],
          acc.at[pl.ds(tid * _ROWS_PT + nz * chunk, zr)], ssem[0])
    pltpu.make_async_copy(src_hbm.at[wid] if not src_1d
                          else src_hbm.at[pl.ds(0, _EPW)], src_v,
                          gsem[0]).wait()
    pltpu.make_async_copy(dst_hbm.at[0], dst_v, gsem[0]).wait()
    for j in range(nz):
      pltpu.make_async_copy(
          rows[0], acc.at[pl.ds(tid * _ROWS_PT + j * chunk, chunk)],
          ssem[0]).wait()
    if zr:
      pltpu.make_async_copy(
          rows[0].at[pl.ds(0, zr)],
          acc.at[pl.ds(tid * _ROWS_PT + nz * chunk, zr)], ssem[0]).wait()
    if with_counts:
      def zc(i, c):
        zcnt_v[pl.ds(i * 16, 16)] = jnp.zeros((16,), jnp.float32)
        return c
      lax.fori_loop(0, _ROWS_PT // 16, zc, 0)
      pltpu.sync_copy(zcnt_v, cnt_acc.at[pl.ds(tid * _ROWS_PT, _ROWS_PT)])
      for i in range(8):
        ones_v[pl.ds(i * 16, 16)] = jnp.ones((16,), jnp.float32)

    def fire_g(k, q):
      idx = (src_v.at[pl.ds(k * chunk, chunk)] if src_1d else src_v.at[k])
      pltpu.async_copy(x_hbm.at[idx], rows[q], gsem[q])

    def wait_g(q):
      pltpu.make_async_copy(
          x_hbm.at[pl.ds(0, chunk)], rows[q], gsem[q]).wait()

    def fire_s(k, q):
      if async_s:
        pltpu.async_copy(rows[q], acc.at[dst_v.at[k]], ssem[q], add=True)
      else:
        pltpu.sync_copy(rows[q], acc.at[dst_v.at[k]], add=True)
      if with_counts:
        pltpu.sync_copy(ones_v.at[pl.ds(0, chunk)],
                        cnt_acc.at[dst_v.at[k]], add=True)

    def wait_s(q):
      if async_s:
        pltpu.make_async_copy(
            x_hbm.at[pl.ds(0, chunk)], rows[q], ssem[q]).wait()

    for q in range(nbuf):
      fire_g(q, q)
    plsc.subcore_barrier()

    full = nch // nbuf
    def grp(kg, c):
      c0 = kg * nbuf
      if async_s:
        for q in range(nbuf):
          wait_g(q)
          fire_s(c0 + q, q)
        for q in range(nbuf):
          nxt = c0 + nbuf + q
          @pl.when(nxt < nch)
          def _():
            wait_s(q)
            fire_g(nxt, q)
      else:
        for q in range(nbuf):
          wait_g(q)
          fire_s(c0 + q, q)
          nxt = c0 + nbuf + q
          @pl.when(nxt < nch)
          def _():
            fire_g(nxt, q)
      return c
    lax.fori_loop(0, full, grp, 0)
    for q in range(nch - full * nbuf):
      wait_g(q)
      fire_s(full * nbuf + q, q)
    for q in range(nbuf):
      wait_s(q)

    plsc.subcore_barrier()
    row0 = tid * _ROWS_PT
    pltpu.async_copy(acc.at[pl.ds(row0, _ROWS_PT)],
                     out_hbm.at[cid, pl.ds(row0, _ROWS_PT)], gsem[0])
    if with_counts:
      pltpu.async_copy(cnt_acc.at[pl.ds(row0, _ROWS_PT)],
                       cnt_hbm.at[pl.ds(cid * _NP + row0, _ROWS_PT)], gsem[0])
    pltpu.make_async_copy(acc.at[pl.ds(row0, _ROWS_PT)],
                          out_hbm.at[cid, pl.ds(row0, _ROWS_PT)],
                          gsem[0]).wait()
    if with_counts:
      pltpu.make_async_copy(cnt_acc.at[pl.ds(row0, _ROWS_PT)],
                            cnt_hbm.at[pl.ds(cid * _NP + row0, _ROWS_PT)],
                            gsem[0]).wait()

  untiled = d == 64 or dt == jnp.bfloat16
  params = pltpu.CompilerParams(use_tc_tiling_on_sc=False) if untiled else None
  return pl.kernel(body, out_type=tuple(out_type), mesh=mesh,
                   scratch_types=tuple(scratch), compiler_params=params)


_BN = 2048


def _inv_of(cnt_blk):
  c = cnt_blk[0] + cnt_blk[1]
  return (1.0 / jnp.maximum(c, 1.0))[:, None]


def _psum(p_ref):
  return (p_ref[0].astype(jnp.float32) + p_ref[1].astype(jnp.float32))


def _dot(a, b):
  return jax.lax.dot_general(a, b, (((1,), (0,)), ((), ())),
                             preferred_element_type=jnp.float32)


def _tc_call(body, out_shapes, in_specs, out_specs):
  return pl.pallas_call(
      body,
      grid=(_NP // _BN,),
      in_specs=in_specs,
      out_specs=out_specs,
      out_shape=out_shapes,
  )


def _spec_rows(d):
  return pl.BlockSpec((_BN, d), lambda i: (i, 0))


def _spec_pair(d):
  return pl.BlockSpec((2, _BN, d), lambda i: (0, i, 0))


def _spec_cnt():
  return pl.BlockSpec((2, _BN), lambda i: (0, i))


def _spec_full(r, c):
  return pl.BlockSpec((r, c), lambda i: (0, 0))


def _stage_a(p0, cnt, x, wl0t, bl0, wr0t, wl1t):
  def body(p_ref, c_ref, x_ref, wl_ref, b_ref, wr_ref, w1_ref, h0_ref, t1_ref):
    m = _psum(p_ref) * _inv_of(c_ref)
    h0 = jnp.maximum(
        _dot(m, wl_ref[...]) + b_ref[...] + _dot(x_ref[...], wr_ref[...]), 0.0)
    h0_ref[...] = h0
    t1_ref[...] = _dot(h0, w1_ref[...])
  return _tc_call(
      body,
      (jax.ShapeDtypeStruct((_NP, 128), jnp.float32),
       jax.ShapeDtypeStruct((_NP, 64), jnp.float32)),
      [_spec_pair(128), _spec_cnt(), _spec_rows(128), _spec_full(128, 128),
       _spec_full(1, 128), _spec_full(128, 128), _spec_full(128, 64)],
      (_spec_rows(128), _spec_rows(64)),
  )(p0, cnt, x, wl0t, bl0, wr0t, wl1t)


def _stage_b(p1, cnt, h0, bl1, wr1t):
  def body(p_ref, c_ref, h_ref, b_ref, wr_ref, o_ref):
    m = _psum(p_ref) * _inv_of(c_ref)
    o_ref[...] = m + b_ref[...] + _dot(h_ref[...], wr_ref[...])
  return _tc_call(
      body,
      jax.ShapeDtypeStruct((_NP, 64), jnp.float32),
      [_spec_pair(64), _spec_cnt(), _spec_rows(128), _spec_full(1, 64),
       _spec_full(128, 64)],
      _spec_rows(64),
  )(p1, cnt, h0, bl1, wr1t)


def _stage_c(p2, cnt, h1, wl2t, bl2, wr2t):
  def body(p_ref, c_ref, h_ref, wl_ref, b_ref, wr_ref, o_ref, ob_ref):
    m = _psum(p_ref) * _inv_of(c_ref)
    h2 = jnp.maximum(
        _dot(m, wl_ref[...]) + b_ref[...] + _dot(h_ref[...], wr_ref[...]), 0.0)
    o_ref[...] = h2
    ob_ref[...] = h2.astype(jnp.bfloat16)
  return _tc_call(
      body,
      (jax.ShapeDtypeStruct((_NP, 128), jnp.float32),
       jax.ShapeDtypeStruct((_NP, 128), jnp.bfloat16)),
      [_spec_pair(64), _spec_cnt(), _spec_rows(64), _spec_full(64, 128),
       _spec_full(1, 128), _spec_full(64, 128)],
      (_spec_rows(128), _spec_rows(128)),
  )(p2, cnt, h1, wl2t, bl2, wr2t)


def _stage_d(p3, cnt, h2, wl3t, bl3, wr3t):
  def body(p_ref, c_ref, h_ref, wl_ref, b_ref, wr_ref, o_ref):
    m = _psum(p_ref) * _inv_of(c_ref)
    o_ref[...] = (_dot(m, wl_ref[...]) + b_ref[...]
                  + _dot(h_ref[...], wr_ref[...]))
  return _tc_call(
      body,
      jax.ShapeDtypeStruct((_NP, 128), jnp.float32),
      [_spec_pair(128), _spec_cnt(), _spec_rows(128), _spec_full(128, 128),
       _spec_full(1, 128), _spec_full(128, 128)],
      _spec_rows(128),
  )(p3, cnt, h2, wl3t, bl3, wr3t)


def _pad_cols(w, n):
  return jnp.concatenate([w, jnp.zeros((w.shape[0], n), jnp.float32)], axis=1)


def _pad_rows(w, n):
  return jnp.concatenate([w, jnp.zeros((n, w.shape[1]), jnp.float32)], axis=0)


@jax.jit
def _run(x, edge_index, Wl0, bl0, Wr0, Wl1, bl1, Wr1, Wl2, bl2, Wr2,
         Wl3, bl3, Wr3):
  xp = jnp.concatenate(
      [x, jnp.zeros((_NP - _N, 128), jnp.float32)], axis=0)
  src125 = edge_index[0].reshape(_NWORK, _EPW // 125, 125)
  dst125 = edge_index[1].reshape(_NWORK, _EPW // 125, 125)

  agg128c = _make_agg(128, 125, 4, True, dt=jnp.bfloat16)
  agg64 = _make_agg(64, 125, 4, False)
  agg128 = _make_agg(128, 125, 4, False, dt=jnp.bfloat16)

  xbf = xp.astype(jnp.bfloat16)
  p0, cnt = agg128c(xbf, src125, dst125)
  cnt = cnt.reshape(2, _NP)
  h0, t1 = _stage_a(p0, cnt, xp, Wl0.T, bl0.reshape(1, 128), Wr0.T, Wl1.T)
  (p1,) = agg64(t1, src125, dst125)
  h1 = _stage_b(p1, cnt, h0, bl1.reshape(1, 64), Wr1.T)
  (p2,) = agg64(h1, src125, dst125)
  h2, h2bf = _stage_c(p2, cnt, h1, Wl2.T, bl2.reshape(1, 128), Wr2.T)
  (p3,) = agg128(h2bf, src125, dst125)
  out = _stage_d(p3, cnt, h2, Wl3.T, bl3.reshape(1, 128), Wr3.T)
  return out[:_N]


def kernel(x, edge_index, Wl0, bl0, Wr0, Wl1, bl1, Wr1, Wl2, bl2, Wr2,
           Wl3, bl3, Wr3):
  return _run(x, edge_index, Wl0, bl0, Wr0, Wl1, bl1, Wr1, Wl2, bl2, Wr2,
              Wl3, bl3, Wr3)
